# baseline (speedup 1.0000x reference)
import jax
import jax.numpy as jnp
from jax import lax
from jax.experimental import pallas as pl
from jax.experimental.pallas import tpu as pltpu
from jax.experimental.pallas import tpu_sc as plsc

B, L, V = 1024, 50, 1000
B_TC = 512
B_SC = B - B_TC
_INFO = plsc.get_sparse_core_info()
NC, NS, LANES = _INFO.num_cores, _INFO.num_subcores, _INFO.num_lanes
NW = NC * NS
RPW = B_SC // NW


def _sc_body(x_hbm, out_hbm, xv, outv, tb):
    wid = lax.axis_index("s") * NC + lax.axis_index("c")
    base = wid * (RPW * L)
    pltpu.sync_copy(x_hbm.at[pl.ds(base, RPW * L)], xv)

    rows = lax.broadcasted_iota(jnp.int32, (LANES,), 0)
    zeros = jnp.zeros((LANES,), jnp.int32)
    ones = jnp.ones((LANES,), jnp.int32)

    @plsc.parallel_loop(0, L, unroll=5)
    def _(l):
        toks = plsc.load_gather(xv, [rows * L + l])
        plsc.store_scatter(tb, [rows * V + toks], zeros)

    def count_body(l, carry):
        toks = plsc.load_gather(xv, [rows * L + l])
        idx = rows * V + toks
        plsc.addupdate_scatter(tb, [idx], ones)
        c = plsc.load_gather(tb, [idx])
        plsc.store_scatter(outv, [rows * L + l], c.astype(jnp.float32))
        return carry

    lax.fori_loop(0, L, count_body, 0, unroll=5)

    pltpu.sync_copy(outv, out_hbm.at[pl.ds(base, RPW * L)])


def _tc_body(x_ref, o_ref):
    x = x_ref[...]
    pos = lax.broadcasted_iota(jnp.int32, (B_TC, L), 1)
    acc = jnp.zeros((B_TC, L), jnp.float32)
    for j in range(L):
        eq = (x == x[:, j : j + 1]) & (pos >= j)
        acc = acc + eq.astype(jnp.float32)
    o_ref[...] = acc


def kernel(x):
    x = x.astype(jnp.int32)

    sc = pl.kernel(
        _sc_body,
        out_type=jax.ShapeDtypeStruct((B_SC * L,), jnp.float32),
        mesh=plsc.VectorSubcoreMesh(core_axis_name="c", subcore_axis_name="s"),
        compiler_params=pltpu.CompilerParams(
            needs_layout_passes=False,
            disable_bounds_checks=True,
            disable_semaphore_checks=True,
        ),
        scratch_types=[
            pltpu.VMEM((RPW * L,), jnp.int32),
            pltpu.VMEM((RPW * L,), jnp.float32),
            pltpu.VMEM((RPW * V,), jnp.int32),
        ],
    )
    sc_out = sc(x[B_TC:].reshape(B_SC * L)).reshape(B_SC, L)

    tc_out = pl.pallas_call(
        _tc_body,
        out_shape=jax.ShapeDtypeStruct((B_TC, L), jnp.float32),
        grid=(1,),
        in_specs=[pl.BlockSpec((B_TC, L), lambda i: (0, 0))],
        out_specs=pl.BlockSpec((B_TC, L), lambda i: (0, 0)),
    )(x)

    return jnp.concatenate([tc_out, sc_out], axis=0)[..., None]


# speedup vs baseline: 1.1937x; 1.1937x over previous
import jax
import jax.numpy as jnp
from jax import lax
from jax.experimental import pallas as pl
from jax.experimental.pallas import tpu as pltpu
from jax.experimental.pallas import tpu_sc as plsc

B, L, V = 1024, 50, 1000
_INFO = plsc.get_sparse_core_info()
NC, NS, LANES = _INFO.num_cores, _INFO.num_subcores, _INFO.num_lanes
NW = NC * NS
RPW = B // NW
NG = RPW // LANES


def _sc_body(x_hbm, out_hbm, xv, outv, tb):
    wid = lax.axis_index("s") * NC + lax.axis_index("c")
    base = wid * (RPW * L)
    pltpu.sync_copy(x_hbm.at[pl.ds(base, RPW * L)], xv)

    lane = lax.broadcasted_iota(jnp.int32, (LANES,), 0)
    zeros = jnp.zeros((LANES,), jnp.int32)
    ones = jnp.ones((LANES,), jnp.int32)
    rows_g = [lane + g * LANES for g in range(NG)]

    @plsc.parallel_loop(0, L, unroll=5)
    def _(l):
        for g in range(NG):
            toks = plsc.load_gather(xv, [rows_g[g] * L + l])
            plsc.store_scatter(tb, [rows_g[g] * V + toks], zeros)

    def count_body(l, carry):
        for g in range(NG):
            toks = plsc.load_gather(xv, [rows_g[g] * L + l])
            idx = rows_g[g] * V + toks
            c = plsc.load_gather(tb, [idx]) + ones
            plsc.store_scatter(tb, [idx], c)
            plsc.store_scatter(outv, [rows_g[g] * L + l], c.astype(jnp.float32))
        return carry

    lax.fori_loop(0, L, count_body, 0, unroll=5)

    pltpu.sync_copy(outv, out_hbm.at[pl.ds(base, RPW * L)])


def kernel(x):
    f = pl.kernel(
        _sc_body,
        out_type=jax.ShapeDtypeStruct((B * L,), jnp.float32),
        mesh=plsc.VectorSubcoreMesh(core_axis_name="c", subcore_axis_name="s"),
        compiler_params=pltpu.CompilerParams(
            needs_layout_passes=False,
            disable_bounds_checks=True,
            disable_semaphore_checks=True,
        ),
        scratch_types=[
            pltpu.VMEM((RPW * L,), jnp.int32),
            pltpu.VMEM((RPW * L,), jnp.float32),
            pltpu.VMEM((RPW * V,), jnp.int32),
        ],
    )
    out = f(x.astype(jnp.int32).reshape(B * L))
    return out.reshape(B, L, 1)


# speedup vs baseline: 1.1976x; 1.0033x over previous
import jax
import jax.numpy as jnp
from jax import lax
from jax.experimental import pallas as pl
from jax.experimental.pallas import tpu as pltpu
from jax.experimental.pallas import tpu_sc as plsc

B, L, V = 1024, 50, 1000
_INFO = plsc.get_sparse_core_info()
NC, NS, LANES = _INFO.num_cores, _INFO.num_subcores, _INFO.num_lanes
NW = NC * NS
RPW = B // NW
NG = RPW // LANES


def _sc_body(x_hbm, out_hbm, xv, outv, tb):
    wid = lax.axis_index("s") * NC + lax.axis_index("c")
    base = wid * (RPW * L)
    pltpu.sync_copy(x_hbm.at[pl.ds(base, RPW * L)], xv)

    lane = lax.broadcasted_iota(jnp.int32, (LANES,), 0)
    zeros = jnp.zeros((LANES,), jnp.int32)
    ones = jnp.ones((LANES,), jnp.int32)
    rows_g = [lane + g * LANES for g in range(NG)]

    @plsc.parallel_loop(0, L, unroll=5)
    def _(l):
        for g in range(NG):
            toks = plsc.load_gather(xv, [rows_g[g] * L + l])
            plsc.store_scatter(tb, [rows_g[g] * V + toks], zeros)

    def count_body(l, carry):
        for g in range(NG):
            toks = plsc.load_gather(xv, [rows_g[g] * L + l])
            idx = rows_g[g] * V + toks
            c = plsc.load_gather(tb, [idx]) + ones
            plsc.store_scatter(tb, [idx], c)
            plsc.store_scatter(outv, [rows_g[g] * L + l], c.astype(jnp.float32))
        return carry

    lax.fori_loop(0, L, count_body, 0, unroll=5)

    pltpu.sync_copy(outv, out_hbm.at[pl.ds(base, RPW * L)])


def kernel(x):
    f = pl.kernel(
        _sc_body,
        out_type=jax.ShapeDtypeStruct((B * L,), jnp.float32),
        mesh=plsc.VectorSubcoreMesh(core_axis_name="c", subcore_axis_name="s"),
        compiler_params=pltpu.CompilerParams(
            needs_layout_passes=False,
            disable_bounds_checks=True,
            disable_semaphore_checks=True,
            allow_input_fusion=[True],
        ),
        scratch_types=[
            pltpu.VMEM((RPW * L,), jnp.int32),
            pltpu.VMEM((RPW * L,), jnp.float32),
            pltpu.VMEM((RPW * V,), jnp.int32),
        ],
    )
    out = f(x.astype(jnp.int32).reshape(B * L))
    return out.reshape(B, L, 1)
